# baseline (speedup 1.0000x reference)
import math

import jax
import jax.numpy as jnp
from jax import lax
from jax.experimental import pallas as pl
from jax.experimental.pallas import tpu as pltpu

B = 4
N = 1024
C = 256
K = 512
KU = 128
ITERS = 3
TAU_GATE = 0.7
TAU_REFINE = 1.0
EPS = 1e-8
JITTER = 1e-6


def _mm(a, b, ca, cb):
    return lax.dot_general(a, b, (((ca,), (cb,)), ((), ())),
                           preferred_element_type=jnp.float32)


def _outer9(v):
    return jnp.concatenate(
        [v * v[:, 0:1], v * v[:, 1:2], v * v[:, 2:3]], axis=1)


def _body(s_ref, mu_ref, sig9_ref, mrow_ref, mcol_ref, pool_ref,
          wg_ref, wq_ref, wk_ref, wi_ref, wh_ref, bi_ref, bh_ref,
          lng_ref, lnb_ref, w1_ref, b1_ref, w2_ref, b2_ref,
          wgeo_ref, gsr_ref, gb_ref,
          A_ref, slots_ref, muk_ref, sigk_ref, idx_ref, sim_scr):
    s = s_ref[0]
    mu = mu_ref[0]
    sig9 = sig9_ref[0]
    mrow = mrow_ref[0]
    mcol = mcol_ref[0]
    pool = pool_ref[...]

    t = _mm(s, wg_ref[...], 1, 1)
    tn = t / jnp.maximum(
        jnp.sqrt(jnp.sum(t * t, axis=1, keepdims=True)), EPS)
    bn = pool / jnp.maximum(
        jnp.sqrt(jnp.sum(pool * pool, axis=1, keepdims=True)), EPS)
    simT = (_mm(bn, tn, 1, 1) / TAU_GATE) * mrow
    sim_scr[...] = simT

    iota_k = lax.broadcasted_iota(jnp.int32, (K, 1), 0)
    iota_u = lax.broadcasted_iota(jnp.int32, (1, KU), 1)

    def gstep(i, carry):
        covered, picked, idxrow = carry
        gain = jnp.maximum(covered, simT) - covered
        gs = jnp.sum(gain, axis=1, keepdims=True)
        gs = jnp.where(picked > 0.5, -1e9, gs)
        gmax = jnp.max(gs)
        cand = jnp.where(gs == gmax, iota_k, K)
        nxt = jnp.min(cand)
        picked = jnp.where(iota_k == nxt, 1.0, picked)
        idxrow = jnp.where(iota_u == i, nxt, idxrow)
        best = sim_scr[pl.ds(nxt, 1), :]
        covered = jnp.maximum(covered, best)
        return covered, picked, idxrow

    covered0 = jnp.full((1, N), -1e9, jnp.float32)
    picked0 = jnp.zeros((K, 1), jnp.float32)
    idxrow0 = jnp.zeros((1, KU), jnp.int32)
    _, _, idxrow = lax.fori_loop(0, KU, gstep,
                                 (covered0, picked0, idxrow0), unroll=32)

    P = (iota_k == idxrow).astype(jnp.float32)
    sim_sel = _mm(simT, P, 0, 0)
    slots = _mm(P, pool, 0, 0)

    def softmax_lanes(x):
        m = jnp.max(x, axis=1, keepdims=True)
        e = jnp.exp(x - m)
        return e / jnp.sum(e, axis=1, keepdims=True)

    A = softmax_lanes(sim_sel) * mcol

    mumu9 = _outer9(mu)
    ones3 = jnp.ones((1, 3), jnp.float32)
    l9 = lax.broadcasted_iota(jnp.int32, (1, 9), 1)
    eye9 = jnp.where(l9 % 4 == 0, JITTER, 0.0)

    def merge(A):
        Aw = A * mcol
        occ = jnp.maximum(jnp.sum(Aw, axis=0, keepdims=True), EPS)
        occ_c = _mm(occ, jnp.ones((1, 1), jnp.float32), 0, 0)
        occ_c = jnp.maximum(occ_c, EPS)
        mu_k = _mm(Aw, mu, 0, 0) / occ_c
        e1 = _mm(Aw, sig9, 0, 0)
        m2 = _mm(Aw, mumu9, 0, 0)
        sig_k = (e1 + m2 - occ_c * _outer9(mu_k)) / occ_c + eye9
        return mu_k, sig_k, occ_c

    mu_k, sig_k, _ = merge(A)

    sbar = jnp.mean(jax.nn.softplus(gsr_ref[...]))
    bbar = jnp.mean(gb_ref[...])
    wgeo = wgeo_ref[0, 0]
    inv_sqrt_c = 1.0 / (math.sqrt(C) * TAU_REFINE)
    kk = _mm(s, wk_ref[...], 1, 1)
    mu2 = jnp.sum(mu * mu, axis=1, keepdims=True)

    for _ in range(ITERS):
        q = _mm(slots, wq_ref[...], 1, 1)
        logits = _mm(kk, q, 1, 1) * inv_sqrt_c
        muk2_row = _mm(ones3, mu_k * mu_k, 1, 1)
        cross = _mm(mu, mu_k, 1, 1)
        dist2 = mu2 - 2.0 * cross + muk2_row
        geo = -sbar * dist2 + bbar
        logits = logits + wgeo * geo
        logits = jnp.where(mcol < 0.5, -1e9, logits)
        A = softmax_lanes(logits) * mcol
        denom = jnp.maximum(jnp.sum(A, axis=0, keepdims=True), EPS)
        denom_c = jnp.maximum(
            _mm(denom, jnp.ones((1, 1), jnp.float32), 0, 0), EPS)
        upd = _mm(A, s, 0, 0) / denom_c
        gi = _mm(upd, wi_ref[...], 1, 1) + bi_ref[...]
        gh = _mm(slots, wh_ref[...], 1, 1) + bh_ref[...]
        r = jax.nn.sigmoid(gi[:, :C] + gh[:, :C])
        z = jax.nn.sigmoid(gi[:, C:2 * C] + gh[:, C:2 * C])
        nn_ = jnp.tanh(gi[:, 2 * C:] + r * gh[:, 2 * C:])
        slots = (1.0 - z) * nn_ + z * slots
        mean = jnp.mean(slots, axis=1, keepdims=True)
        var = jnp.mean((slots - mean) ** 2, axis=1, keepdims=True)
        ln = (slots - mean) / jnp.sqrt(var + 1e-5) * lng_ref[...] \
            + lnb_ref[...]
        h1 = jax.nn.gelu(_mm(ln, w1_ref[...], 1, 1) + b1_ref[...])
        slots = slots + _mm(h1, w2_ref[...], 1, 1) + b2_ref[...]
        mu_k, sig_k, _ = merge(A)

    A_ref[0] = A
    slots_ref[0] = slots
    muk_ref[0] = mu_k
    sigk_ref[0] = sig_k
    idx_ref[0] = idxrow


def kernel(s, mu, Sigma, mask, slot_embed_pool, W_gate, W_q, W_k,
           gru_Wi, gru_Wh, gru_bi, gru_bh, ln_g, ln_b,
           W_mlp1, b_mlp1, W_mlp2, b_mlp2, w_geo, geo_scale_raw, geo_bias):
    sig9 = Sigma.reshape(B, N, 9)
    mrow = mask.reshape(B, 1, N)
    mcol = mask.reshape(B, N, 1)

    def bcast(shape):
        nd = len(shape)
        return pl.BlockSpec(shape, lambda i, _n=nd: (0,) * _n)

    def per_b(shape):
        nd = len(shape)
        return pl.BlockSpec((1,) + shape,
                            lambda i, _n=nd: (i,) + (0,) * _n)

    in_specs = [
        per_b((N, C)),
        per_b((N, 3)),
        per_b((N, 9)),
        per_b((1, N)),
        per_b((N, 1)),
        bcast((K, C)),
        bcast((C, C)),
        bcast((C, C)),
        bcast((C, C)),
        bcast((3 * C, C)),
        bcast((3 * C, C)),
        bcast((1, 3 * C)),
        bcast((1, 3 * C)),
        bcast((1, C)),
        bcast((1, C)),
        bcast((4 * C, C)),
        bcast((1, 4 * C)),
        bcast((C, 4 * C)),
        bcast((1, C)),
        bcast((1, 1)),
        bcast((1, 4)),
        bcast((1, 4)),
    ]
    out_specs = [
        per_b((N, KU)),
        per_b((KU, C)),
        per_b((KU, 3)),
        per_b((KU, 9)),
        per_b((1, KU)),
    ]
    out_shape = [
        jax.ShapeDtypeStruct((B, N, KU), jnp.float32),
        jax.ShapeDtypeStruct((B, KU, C), jnp.float32),
        jax.ShapeDtypeStruct((B, KU, 3), jnp.float32),
        jax.ShapeDtypeStruct((B, KU, 9), jnp.float32),
        jax.ShapeDtypeStruct((B, 1, KU), jnp.int32),
    ]
    A, slots, mu_k, sig_k9, idx3 = pl.pallas_call(
        _body,
        grid=(B,),
        compiler_params=pltpu.CompilerParams(
            dimension_semantics=("parallel",)),
        in_specs=in_specs,
        out_specs=out_specs,
        out_shape=out_shape,
        scratch_shapes=[pltpu.VMEM((K, N), jnp.float32)],
    )(s, mu, sig9, mrow, mcol, slot_embed_pool, W_gate, W_q, W_k,
      gru_Wi, gru_Wh, gru_bi.reshape(1, -1), gru_bh.reshape(1, -1),
      ln_g.reshape(1, -1), ln_b.reshape(1, -1), W_mlp1,
      b_mlp1.reshape(1, -1), W_mlp2, b_mlp2.reshape(1, -1),
      jnp.reshape(w_geo, (1, 1)), geo_scale_raw.reshape(1, -1),
      geo_bias.reshape(1, -1))
    return (A, slots, mu_k, sig_k9.reshape(B, KU, 3, 3),
            idx3.reshape(B, KU))


# speedup vs baseline: 1.0060x; 1.0060x over previous
import math

import jax
import jax.numpy as jnp
from jax import lax
from jax.experimental import pallas as pl
from jax.experimental.pallas import tpu as pltpu

B = 4
N = 1024
C = 256
K = 512
KU = 128
ITERS = 3
TAU_GATE = 0.7
TAU_REFINE = 1.0
EPS = 1e-8
JITTER = 1e-6


def _mm(a, b, ca, cb):
    return lax.dot_general(a, b, (((ca,), (cb,)), ((), ())),
                           preferred_element_type=jnp.float32)


def _outer9(v):
    return jnp.concatenate(
        [v * v[:, 0:1], v * v[:, 1:2], v * v[:, 2:3]], axis=1)


def _body(s_ref, mu_ref, sig9_ref, mrow_ref, mcol_ref, pool_ref,
          wg_ref, wq_ref, wk_ref, wi_ref, wh_ref, bi_ref, bh_ref,
          lng_ref, lnb_ref, w1_ref, b1_ref, w2_ref, b2_ref,
          wgeo_ref, gsr_ref, gb_ref,
          A_ref, slots_ref, muk_ref, sigk_ref, idx_ref, sim_scr):
    s = s_ref[0]
    mu = mu_ref[0]
    sig9 = sig9_ref[0]
    mrow = mrow_ref[0]
    mcol = mcol_ref[0]
    pool = pool_ref[...]

    t = _mm(s, wg_ref[...], 1, 1)
    tn = t / jnp.maximum(
        jnp.sqrt(jnp.sum(t * t, axis=1, keepdims=True)), EPS)
    bn = pool / jnp.maximum(
        jnp.sqrt(jnp.sum(pool * pool, axis=1, keepdims=True)), EPS)
    simT = (_mm(bn, tn, 1, 1) / TAU_GATE) * mrow
    sim_scr[...] = simT

    iota_k = lax.broadcasted_iota(jnp.int32, (K, 1), 0)
    iota_u = lax.broadcasted_iota(jnp.int32, (1, KU), 1)

    def gstep(i, carry):
        covered, picked, idxrow = carry
        gain = jnp.maximum(covered, simT) - covered
        gs = jnp.sum(gain, axis=1, keepdims=True)
        gs = jnp.where(picked > 0.5, -1e9, gs)
        gmax = jnp.max(gs)
        cand = jnp.where(gs == gmax, iota_k, K)
        nxt = jnp.min(cand)
        picked = jnp.where(iota_k == nxt, 1.0, picked)
        idxrow = jnp.where(iota_u == i, nxt, idxrow)
        best = sim_scr[pl.ds(nxt, 1), :]
        covered = jnp.maximum(covered, best)
        return covered, picked, idxrow

    covered0 = jnp.full((1, N), -1e9, jnp.float32)
    picked0 = jnp.zeros((K, 1), jnp.float32)
    idxrow0 = jnp.zeros((1, KU), jnp.int32)
    _, _, idxrow = lax.fori_loop(0, KU, gstep,
                                 (covered0, picked0, idxrow0), unroll=16)

    P = (iota_k == idxrow).astype(jnp.float32)
    sim_sel = _mm(simT, P, 0, 0)
    slots = _mm(P, pool, 0, 0)

    def softmax_lanes(x):
        m = jnp.max(x, axis=1, keepdims=True)
        e = jnp.exp(x - m)
        return e / jnp.sum(e, axis=1, keepdims=True)

    A = softmax_lanes(sim_sel) * mcol

    mumu9 = _outer9(mu)
    ones3 = jnp.ones((1, 3), jnp.float32)
    l9 = lax.broadcasted_iota(jnp.int32, (1, 9), 1)
    eye9 = jnp.where(l9 % 4 == 0, JITTER, 0.0)

    def merge(A):
        Aw = A * mcol
        occ = jnp.maximum(jnp.sum(Aw, axis=0, keepdims=True), EPS)
        occ_c = _mm(occ, jnp.ones((1, 1), jnp.float32), 0, 0)
        occ_c = jnp.maximum(occ_c, EPS)
        mu_k = _mm(Aw, mu, 0, 0) / occ_c
        e1 = _mm(Aw, sig9, 0, 0)
        m2 = _mm(Aw, mumu9, 0, 0)
        sig_k = (e1 + m2 - occ_c * _outer9(mu_k)) / occ_c + eye9
        return mu_k, sig_k, occ_c

    mu_k, sig_k, _ = merge(A)

    sbar = jnp.mean(jax.nn.softplus(gsr_ref[...]))
    bbar = jnp.mean(gb_ref[...])
    wgeo = wgeo_ref[0, 0]
    inv_sqrt_c = 1.0 / (math.sqrt(C) * TAU_REFINE)
    kk = _mm(s, wk_ref[...], 1, 1)
    mu2 = jnp.sum(mu * mu, axis=1, keepdims=True)

    for _ in range(ITERS):
        q = _mm(slots, wq_ref[...], 1, 1)
        logits = _mm(kk, q, 1, 1) * inv_sqrt_c
        muk2_row = _mm(ones3, mu_k * mu_k, 1, 1)
        cross = _mm(mu, mu_k, 1, 1)
        dist2 = mu2 - 2.0 * cross + muk2_row
        geo = -sbar * dist2 + bbar
        logits = logits + wgeo * geo
        logits = jnp.where(mcol < 0.5, -1e9, logits)
        A = softmax_lanes(logits) * mcol
        denom = jnp.maximum(jnp.sum(A, axis=0, keepdims=True), EPS)
        denom_c = jnp.maximum(
            _mm(denom, jnp.ones((1, 1), jnp.float32), 0, 0), EPS)
        upd = _mm(A, s, 0, 0) / denom_c
        gi = _mm(upd, wi_ref[...], 1, 1) + bi_ref[...]
        gh = _mm(slots, wh_ref[...], 1, 1) + bh_ref[...]
        r = jax.nn.sigmoid(gi[:, :C] + gh[:, :C])
        z = jax.nn.sigmoid(gi[:, C:2 * C] + gh[:, C:2 * C])
        nn_ = jnp.tanh(gi[:, 2 * C:] + r * gh[:, 2 * C:])
        slots = (1.0 - z) * nn_ + z * slots
        mean = jnp.mean(slots, axis=1, keepdims=True)
        var = jnp.mean((slots - mean) ** 2, axis=1, keepdims=True)
        ln = (slots - mean) / jnp.sqrt(var + 1e-5) * lng_ref[...] \
            + lnb_ref[...]
        h1 = jax.nn.gelu(_mm(ln, w1_ref[...], 1, 1) + b1_ref[...])
        slots = slots + _mm(h1, w2_ref[...], 1, 1) + b2_ref[...]
        mu_k, sig_k, _ = merge(A)

    A_ref[0] = A
    slots_ref[0] = slots
    muk_ref[0] = mu_k
    sigk_ref[0] = sig_k
    idx_ref[0] = idxrow


def kernel(s, mu, Sigma, mask, slot_embed_pool, W_gate, W_q, W_k,
           gru_Wi, gru_Wh, gru_bi, gru_bh, ln_g, ln_b,
           W_mlp1, b_mlp1, W_mlp2, b_mlp2, w_geo, geo_scale_raw, geo_bias):
    sig9 = Sigma.reshape(B, N, 9)
    mrow = mask.reshape(B, 1, N)
    mcol = mask.reshape(B, N, 1)

    def bcast(shape):
        nd = len(shape)
        return pl.BlockSpec(shape, lambda i, _n=nd: (0,) * _n)

    def per_b(shape):
        nd = len(shape)
        return pl.BlockSpec((1,) + shape,
                            lambda i, _n=nd: (i,) + (0,) * _n)

    in_specs = [
        per_b((N, C)),
        per_b((N, 3)),
        per_b((N, 9)),
        per_b((1, N)),
        per_b((N, 1)),
        bcast((K, C)),
        bcast((C, C)),
        bcast((C, C)),
        bcast((C, C)),
        bcast((3 * C, C)),
        bcast((3 * C, C)),
        bcast((1, 3 * C)),
        bcast((1, 3 * C)),
        bcast((1, C)),
        bcast((1, C)),
        bcast((4 * C, C)),
        bcast((1, 4 * C)),
        bcast((C, 4 * C)),
        bcast((1, C)),
        bcast((1, 1)),
        bcast((1, 4)),
        bcast((1, 4)),
    ]
    out_specs = [
        per_b((N, KU)),
        per_b((KU, C)),
        per_b((KU, 3)),
        per_b((KU, 9)),
        per_b((1, KU)),
    ]
    out_shape = [
        jax.ShapeDtypeStruct((B, N, KU), jnp.float32),
        jax.ShapeDtypeStruct((B, KU, C), jnp.float32),
        jax.ShapeDtypeStruct((B, KU, 3), jnp.float32),
        jax.ShapeDtypeStruct((B, KU, 9), jnp.float32),
        jax.ShapeDtypeStruct((B, 1, KU), jnp.int32),
    ]
    A, slots, mu_k, sig_k9, idx3 = pl.pallas_call(
        _body,
        grid=(B,),
        compiler_params=pltpu.CompilerParams(
            dimension_semantics=("parallel",)),
        in_specs=in_specs,
        out_specs=out_specs,
        out_shape=out_shape,
        scratch_shapes=[pltpu.VMEM((K, N), jnp.float32)],
    )(s, mu, sig9, mrow, mcol, slot_embed_pool, W_gate, W_q, W_k,
      gru_Wi, gru_Wh, gru_bi.reshape(1, -1), gru_bh.reshape(1, -1),
      ln_g.reshape(1, -1), ln_b.reshape(1, -1), W_mlp1,
      b_mlp1.reshape(1, -1), W_mlp2, b_mlp2.reshape(1, -1),
      jnp.reshape(w_geo, (1, 1)), geo_scale_raw.reshape(1, -1),
      geo_bias.reshape(1, -1))
    return (A, slots, mu_k, sig_k9.reshape(B, KU, 3, 3),
            idx3.reshape(B, KU))
